# baseline (speedup 1.0000x reference)
import functools

import jax
import jax.numpy as jnp
from jax import lax
from jax.experimental import pallas as pl
from jax.experimental.pallas import tpu as pltpu

_EPS = 1e-5
_GUARD = 64


def _conv_bn_kernel(x_ref, w_ref, m_ref, g_ref, b_ref, o_ref, conv_ref,
                    sum_ref, sq_ref, ss_ref, *, n_img, hw, n_valid, taps):
    i = pl.program_id(0)
    cout = w_ref.shape[0]
    cin = x_ref.shape[1]

    @pl.when(i == 0)
    def _init():
        sum_ref[...] = jnp.zeros_like(sum_ref)
        sq_ref[...] = jnp.zeros_like(sq_ref)

    @pl.when(i < n_img)
    def _conv_phase():
        vp = x_ref[0]
        vm = vp * m_ref[0:1, :]
        vq = vp * m_ref[1:2, :]
        srcs = {-1: vm, 0: vp, 1: vq}
        conv = jnp.zeros((cout, hw), jnp.float32)
        for k, (s, kj) in enumerate(taps):
            conv += jnp.dot(w_ref[:, k * cin:(k + 1) * cin],
                            srcs[kj][:, _GUARD + s:_GUARD + s + hw],
                            preferred_element_type=jnp.float32)
        conv_ref[i] = conv
        sum_ref[...] += conv
        sq_ref[...] += conv * conv

    @pl.when(i == n_img)
    def _fold_bn():
        inv_m = 1.0 / float(n_valid)
        tot = jnp.sum(sum_ref[...], axis=1, keepdims=True)
        totsq = jnp.sum(sq_ref[...], axis=1, keepdims=True)
        mean = tot * inv_m
        var = jnp.maximum(totsq * inv_m - mean * mean, 0.0)
        inv_std = lax.rsqrt(var + _EPS)
        scale = g_ref[...] * inv_std
        shift = b_ref[...] - mean * scale
        ss_ref[:, 0:1] = scale
        ss_ref[:, 1:2] = shift

    @pl.when(i >= n_img)
    def _bn_phase():
        t = i - n_img
        scale = ss_ref[:, 0:1]
        shift = ss_ref[:, 1:2]
        o_ref[0] = jnp.maximum(conv_ref[t] * scale + shift, 0.0)


def _conv_block(x, weight, gamma, beta):
    n, cin, h, w = x.shape
    cout = weight.shape[0]
    hw = h * w

    xf = jnp.pad(x.reshape(n, cin, hw).astype(jnp.bfloat16),
                 ((0, 0), (0, 0), (_GUARD, _GUARD)))

    w_mat = jnp.transpose(weight, (0, 2, 3, 1)).reshape(cout, 9 * cin)
    w_mat = w_mat.astype(jnp.bfloat16)

    taps = tuple((w * (ki - 1) + (kj - 1), kj - 1)
                 for ki in range(3) for kj in range(3))

    c = jnp.arange(hw + 2 * _GUARD, dtype=jnp.int32)
    m_km1 = (c % w != (_GUARD - 1) % w)
    m_kp1 = (c % w != _GUARD % w)
    mask = jnp.concatenate(
        [jnp.stack([m_km1, m_kp1], axis=0).astype(jnp.bfloat16),
         jnp.ones((6, hw + 2 * _GUARD), jnp.bfloat16)], axis=0)

    g2 = gamma.astype(jnp.float32).reshape(cout, 1)
    b2 = beta.astype(jnp.float32).reshape(cout, 1)

    kern = functools.partial(
        _conv_bn_kernel, n_img=n, hw=hw, n_valid=n * hw, taps=taps)

    out = pl.pallas_call(
        kern,
        out_shape=jax.ShapeDtypeStruct((n, cout, hw), jnp.float32),
        grid=(2 * n,),
        in_specs=[
            pl.BlockSpec((1, cin, hw + 2 * _GUARD),
                         lambda i, n=n: (jnp.minimum(i, n - 1), 0, 0)),
            pl.BlockSpec((cout, 9 * cin), lambda i: (0, 0)),
            pl.BlockSpec((8, hw + 2 * _GUARD), lambda i: (0, 0)),
            pl.BlockSpec((cout, 1), lambda i: (0, 0)),
            pl.BlockSpec((cout, 1), lambda i: (0, 0)),
        ],
        out_specs=pl.BlockSpec(
            (1, cout, hw), lambda i, n=n: (jnp.maximum(i - n, 0), 0, 0)),
        scratch_shapes=[
            pltpu.VMEM((n, cout, hw), jnp.float32),
            pltpu.VMEM((cout, hw), jnp.float32),
            pltpu.VMEM((cout, hw), jnp.float32),
            pltpu.VMEM((cout, 128), jnp.float32),
        ],
        compiler_params=pltpu.CompilerParams(
            dimension_semantics=("arbitrary",),
            vmem_limit_bytes=100 * 1024 * 1024,
        ),
        cost_estimate=pl.CostEstimate(
            flops=2 * cout * 9 * cin * n * hw + 5 * cout * n * hw,
            transcendentals=cout,
            bytes_accessed=x.size * 4 + n * cout * hw * 4,
        ),
    )(xf, w_mat, mask, g2, b2)

    return out.reshape(n, cout, h, w)


def kernel(x, weight, bias, gamma, beta):
    del bias
    return _conv_block(x, weight, gamma, beta)
